# Initial kernel scaffold; baseline (speedup 1.0000x reference)
import jax
import jax.numpy as jnp
from jax.experimental import pallas as pl
from jax.experimental.pallas import tpu as pltpu

_B, _L, _D, _E, _F, _TOPK = 1, 2048, 1024, 8, 4096, 2
_N = _B * _L
_FT = 512
_NF = _F // _FT


def _router_body(x_ref, gw_ref, logits_ref, combine_ref):
    x = x_ref[...]
    gw = gw_ref[...]
    logits = jax.lax.dot_general(
        x, gw, (((1,), (0,)), ((), ())),
        precision=jax.lax.Precision.HIGHEST,
        preferred_element_type=jnp.float32)
    logits_ref[...] = logits
    eidx = jax.lax.broadcasted_iota(jnp.int32, (_N, _E), 1)
    m1 = jnp.max(logits, axis=1, keepdims=True)
    i1 = jnp.min(jnp.where(logits == m1, eidx, _E), axis=1, keepdims=True)
    masked = jnp.where(eidx == i1, -jnp.inf, logits)
    m2 = jnp.max(masked, axis=1, keepdims=True)
    i2 = jnp.min(jnp.where(masked == m2, eidx, _E), axis=1, keepdims=True)
    t = jnp.exp(m2 - m1)
    denom = 1.0 + t
    wa = 1.0 / denom
    wb = t / denom
    combine_ref[...] = (jnp.where(eidx == i1, wa, 0.0)
                        + jnp.where(eidx == i2, wb, 0.0))


def _expert_body(xb_ref, comb_ref, w1_ref, w2_ref, out_ref):
    e = pl.program_id(0)
    f = pl.program_id(1)

    @pl.when((e == 0) & (f == 0))
    def _():
        out_ref[...] = jnp.zeros_like(out_ref)

    xb = xb_ref[...]
    w1t = w1_ref[0].astype(jnp.bfloat16)
    w2t = w2_ref[0].astype(jnp.bfloat16)
    h = jax.lax.dot_general(
        xb, w1t, (((1,), (0,)), ((), ())),
        preferred_element_type=jnp.float32)
    h = h * jax.lax.logistic(h)
    eidx = jax.lax.broadcasted_iota(jnp.int32, (_N, _E), 1)
    c = jnp.sum(jnp.where(eidx == e, comb_ref[...], 0.0), axis=1,
                keepdims=True)
    h = (h * c).astype(jnp.bfloat16)
    out_ref[...] += jax.lax.dot_general(
        h, w2t, (((1,), (0,)), ((), ())),
        preferred_element_type=jnp.float32)


def kernel(x, gate_w, w1, w2):
    x_flat = x.reshape(_N, _D)
    logits, combine = pl.pallas_call(
        _router_body,
        out_shape=(
            jax.ShapeDtypeStruct((_N, _E), jnp.float32),
            jax.ShapeDtypeStruct((_N, _E), jnp.float32),
        ),
    )(x_flat, gate_w)

    xb = x_flat.astype(jnp.bfloat16)
    out = pl.pallas_call(
        _expert_body,
        grid=(_E, _NF),
        in_specs=[
            pl.BlockSpec((_N, _D), lambda e, f: (0, 0)),
            pl.BlockSpec((_N, _E), lambda e, f: (0, 0)),
            pl.BlockSpec((1, _D, _FT), lambda e, f: (e, 0, f)),
            pl.BlockSpec((1, _FT, _D), lambda e, f: (e, f, 0)),
        ],
        out_specs=pl.BlockSpec((_N, _D), lambda e, f: (0, 0)),
        out_shape=jax.ShapeDtypeStruct((_N, _D), jnp.float32),
        compiler_params=pltpu.CompilerParams(
            dimension_semantics=("arbitrary", "arbitrary")),
    )(xb, combine, w1, w2)
    return out.reshape(_B, _L, _D), logits


# speedup vs baseline: 1.2865x; 1.2865x over previous
import jax
import jax.numpy as jnp
from jax.experimental import pallas as pl
from jax.experimental.pallas import tpu as pltpu

_B, _L, _D, _E, _F, _TOPK = 1, 2048, 1024, 8, 4096, 2
_N = _B * _L
_FT = 512
_NF = _F // _FT


def _router_body(x_ref, gw_ref, logits_ref, combine_ref):
    x = x_ref[...]
    gw = gw_ref[...]
    logits = jax.lax.dot_general(
        x, gw, (((1,), (0,)), ((), ())),
        precision=jax.lax.Precision.DEFAULT,
        preferred_element_type=jnp.float32)
    logits_ref[...] = logits
    eidx = jax.lax.broadcasted_iota(jnp.int32, (_N, _E), 1)
    m1 = jnp.max(logits, axis=1, keepdims=True)
    i1 = jnp.min(jnp.where(logits == m1, eidx, _E), axis=1, keepdims=True)
    masked = jnp.where(eidx == i1, -jnp.inf, logits)
    m2 = jnp.max(masked, axis=1, keepdims=True)
    i2 = jnp.min(jnp.where(masked == m2, eidx, _E), axis=1, keepdims=True)
    t = jnp.exp(m2 - m1)
    denom = 1.0 + t
    wa = 1.0 / denom
    wb = t / denom
    combine_ref[...] = (jnp.where(eidx == i1, wa, 0.0)
                        + jnp.where(eidx == i2, wb, 0.0))


def _expert_body(xb_ref, comb_ref, w1_ref, w2_ref, out_ref):
    e = pl.program_id(0)
    f = pl.program_id(1)

    @pl.when((e == 0) & (f == 0))
    def _():
        out_ref[...] = jnp.zeros_like(out_ref)

    xb = xb_ref[...]
    w1t = w1_ref[0].astype(jnp.bfloat16)
    w2t = w2_ref[0].astype(jnp.bfloat16)
    h = jax.lax.dot_general(
        xb, w1t, (((1,), (0,)), ((), ())),
        preferred_element_type=jnp.float32)
    h = h * jax.lax.logistic(h)
    eidx = jax.lax.broadcasted_iota(jnp.int32, (_N, _E), 1)
    c = jnp.sum(jnp.where(eidx == e, comb_ref[...], 0.0), axis=1,
                keepdims=True)
    h = (h * c).astype(jnp.bfloat16)
    out_ref[...] += jax.lax.dot_general(
        h, w2t, (((1,), (0,)), ((), ())),
        preferred_element_type=jnp.float32)


def kernel(x, gate_w, w1, w2):
    x_flat = x.reshape(_N, _D)
    logits, combine = pl.pallas_call(
        _router_body,
        out_shape=(
            jax.ShapeDtypeStruct((_N, _E), jnp.float32),
            jax.ShapeDtypeStruct((_N, _E), jnp.float32),
        ),
    )(x_flat, gate_w)

    xb = x_flat.astype(jnp.bfloat16)
    out = pl.pallas_call(
        _expert_body,
        grid=(_E, _NF),
        in_specs=[
            pl.BlockSpec((_N, _D), lambda e, f: (0, 0)),
            pl.BlockSpec((_N, _E), lambda e, f: (0, 0)),
            pl.BlockSpec((1, _D, _FT), lambda e, f: (e, 0, f)),
            pl.BlockSpec((1, _FT, _D), lambda e, f: (e, f, 0)),
        ],
        out_specs=pl.BlockSpec((_N, _D), lambda e, f: (0, 0)),
        out_shape=jax.ShapeDtypeStruct((_N, _D), jnp.float32),
        compiler_params=pltpu.CompilerParams(
            dimension_semantics=("arbitrary", "arbitrary")),
    )(xb, combine, w1, w2)
    return out.reshape(_B, _L, _D), logits
